# baseline (speedup 1.0000x reference)
import functools
import math

import jax
import jax.numpy as jnp
import numpy as np
from jax import lax
from jax.experimental import pallas as pl
from jax.experimental.pallas import tpu as pltpu
from jax.experimental.pallas import tpu_sc as plsc

VOCAB = 100000
D_MODEL = 1024
BATCH = 4
SEQ = 8192

NUM_CORES = 2
NUM_SUBCORES = 16
NUM_WORKERS = NUM_CORES * NUM_SUBCORES
S_PER_WORKER = SEQ // NUM_WORKERS
CHUNK = 32
N_CHUNKS = S_PER_WORKER // CHUNK
LANES = 16
VECS_PER_ROW = D_MODEL // LANES


def _pe_scaled():
    pos = np.arange(SEQ, dtype=np.float32)[:, None]
    div = np.exp(
        np.arange(0, D_MODEL, 2, dtype=np.float32)
        * (-math.log(10000.0) / D_MODEL)
    )
    pe = np.zeros((SEQ, D_MODEL), dtype=np.float32)
    pe[:, 0::2] = np.sin(pos * div)
    pe[:, 1::2] = np.cos(pos * div)
    return np.asarray(pe * math.sqrt(D_MODEL), dtype=np.float32)


_PE32 = _pe_scaled()
_SCALE = math.sqrt(D_MODEL)


@functools.partial(
    pl.kernel,
    out_type=jax.ShapeDtypeStruct((BATCH, SEQ, D_MODEL), jnp.float32),
    mesh=plsc.VectorSubcoreMesh(core_axis_name="c", subcore_axis_name="s"),
    scratch_types=[
        pltpu.VMEM((CHUNK,), jnp.int32),
        pltpu.VMEM((CHUNK, D_MODEL), jnp.float32),
        pltpu.VMEM((CHUNK, D_MODEL), jnp.float32),
        pltpu.SemaphoreType.DMA,
    ],
)
def _emb_lookup(x_hbm, pe_hbm, table_hbm, out_hbm, idx_v, pe_v, rows_v, sem):
    wid = lax.axis_index("s") * NUM_CORES + lax.axis_index("c")
    base = wid * S_PER_WORKER

    def chunk_step(i, carry):
        s0 = base + i * CHUNK
        pltpu.sync_copy(pe_hbm.at[pl.ds(s0, CHUNK)], pe_v)
        for b in range(BATCH):
            pltpu.sync_copy(x_hbm.at[b, pl.ds(s0, CHUNK)], idx_v)
            pltpu.async_copy(table_hbm.at[idx_v], rows_v, sem).wait()

            def row_fma(r, carry2):
                for j in range(VECS_PER_ROW):
                    sl = pl.ds(j * LANES, LANES)
                    rows_v[r, sl] = rows_v[r, sl] * _SCALE + pe_v[r, sl]
                return carry2

            lax.fori_loop(0, CHUNK, row_fma, 0)
            pltpu.sync_copy(rows_v, out_hbm.at[b, pl.ds(s0, CHUNK)])
        return carry

    lax.fori_loop(0, N_CHUNKS, chunk_step, 0)


def kernel(x, table):
    return _emb_lookup(x.astype(jnp.int32), jnp.asarray(_PE32), table)


# speedup vs baseline: 1.3591x; 1.3591x over previous
import functools
import math

import jax
import jax.numpy as jnp
import numpy as np
from jax import lax
from jax.experimental import pallas as pl
from jax.experimental.pallas import tpu as pltpu
from jax.experimental.pallas import tpu_sc as plsc

VOCAB = 100000
D_MODEL = 1024
BATCH = 4
SEQ = 8192

NUM_CORES = 2
NUM_SUBCORES = 16
NUM_WORKERS = NUM_CORES * NUM_SUBCORES
S_PER_WORKER = SEQ // NUM_WORKERS
CHUNK = 32
N_CHUNKS = S_PER_WORKER // CHUNK
N_ITEMS = N_CHUNKS * BATCH
LANES = 16
VECS_PER_ROW = D_MODEL // LANES


def _pe_scaled():
    pos = np.arange(SEQ, dtype=np.float32)[:, None]
    div = np.exp(
        np.arange(0, D_MODEL, 2, dtype=np.float32)
        * (-math.log(10000.0) / D_MODEL)
    )
    pe = np.zeros((SEQ, D_MODEL), dtype=np.float32)
    pe[:, 0::2] = np.sin(pos * div)
    pe[:, 1::2] = np.cos(pos * div)
    return np.asarray(pe * math.sqrt(D_MODEL), dtype=np.float32)


_PE32 = _pe_scaled()
_SCALE = math.sqrt(D_MODEL)


@functools.partial(
    pl.kernel,
    out_type=jax.ShapeDtypeStruct((BATCH, SEQ, D_MODEL), jnp.float32),
    mesh=plsc.VectorSubcoreMesh(core_axis_name="c", subcore_axis_name="s"),
    scratch_types=[
        pltpu.VMEM((BATCH, S_PER_WORKER), jnp.int32),
        pltpu.VMEM((CHUNK, D_MODEL), jnp.float32),
        pltpu.VMEM((CHUNK, D_MODEL), jnp.float32),
        pltpu.VMEM((CHUNK, D_MODEL), jnp.float32),
        pltpu.SemaphoreType.DMA,
        pltpu.SemaphoreType.DMA,
        pltpu.SemaphoreType.DMA,
        pltpu.SemaphoreType.DMA,
    ],
)
def _emb_lookup(x_hbm, pe_hbm, table_hbm, out_hbm,
                idx_all, pe_v, rows0, rows1,
                gsem0, gsem1, wsem0, wsem1):
    wid = lax.axis_index("s") * NUM_CORES + lax.axis_index("c")
    base = wid * S_PER_WORKER
    bufs = ((rows0, gsem0, wsem0), (rows1, gsem1, wsem1))

    def item_batch(t):
        return t & 3

    def item_s0(t):
        return base + (t >> 2) * CHUNK

    def issue_gather(t, rows, gsem):
        idx = idx_all.at[item_batch(t), pl.ds((t >> 2) * CHUNK, CHUNK)]
        pltpu.async_copy(table_hbm.at[idx], rows, gsem)

    def wait_write(t, rows, wsem):
        pltpu.make_async_copy(
            rows, out_hbm.at[item_batch(t), pl.ds(item_s0(t), CHUNK)], wsem
        ).wait()

    for b in range(BATCH):
        pltpu.sync_copy(x_hbm.at[b, pl.ds(base, S_PER_WORKER)], idx_all.at[b])
    pltpu.sync_copy(pe_hbm.at[pl.ds(base, CHUNK)], pe_v)
    issue_gather(0, rows0, gsem0)

    def step_fn(step, carry):
        for ph in range(2):
            t = step * 2 + ph
            rows, gsem, _ = bufs[ph]
            n_rows, n_gsem, n_wsem = bufs[1 - ph]

            @pl.when(jnp.logical_and(t >= 1, t + 1 < N_ITEMS))
            def _():
                wait_write(t - 1, n_rows, n_wsem)

            @pl.when(t + 1 < N_ITEMS)
            def _():
                issue_gather(t + 1, n_rows, n_gsem)

            pltpu.make_async_copy(
                table_hbm.at[idx_all.at[0, pl.ds(0, CHUNK)]], rows, gsem
            ).wait()

            def row_fma(r, carry2):
                for j in range(VECS_PER_ROW):
                    sl = pl.ds(j * LANES, LANES)
                    rows[r, sl] = rows[r, sl] * _SCALE + pe_v[r, sl]
                return carry2

            lax.fori_loop(0, CHUNK, row_fma, 0)

            _, _, wsem = bufs[ph]
            pltpu.async_copy(
                rows, out_hbm.at[item_batch(t), pl.ds(item_s0(t), CHUNK)], wsem
            )

            @pl.when(jnp.logical_and((t & 3) == 3, t + 1 < N_ITEMS))
            def _():
                pltpu.sync_copy(
                    pe_hbm.at[pl.ds(base + ((t + 1) >> 2) * CHUNK, CHUNK)], pe_v
                )

        return carry

    lax.fori_loop(0, N_ITEMS // 2, step_fn, 0)

    wait_write(N_ITEMS - 2, rows0, wsem0)
    wait_write(N_ITEMS - 1, rows1, wsem1)


def kernel(x, table):
    return _emb_lookup(x.astype(jnp.int32), jnp.asarray(_PE32), table)
